# baseline (speedup 1.0000x reference)
import functools

import jax
import jax.numpy as jnp
from jax import lax
from jax.experimental import pallas as pl
from jax.experimental.pallas import tpu as pltpu
from jax.experimental.pallas import tpu_sc as plsc

NC = 2
NS = 16
NW = NC * NS
L = 16

T, B, F = 4096, 16, 512
ROW = B * F
ZERO_SLOT = ROW
T_PER_W = T // NW
CHUNKS = ROW // L


_TC_BLK = 128


def _sel_body(x_ref, x0_ref, sel_ref, out_ref):
    @pl.when(pl.program_id(0) == 0)
    def _init():
        sel_ref[...] = jnp.zeros((B, F), jnp.int32)

    xb = x_ref[...]
    neq = (xb != x0_ref[...]).any(axis=0)
    sel_ref[...] = sel_ref[...] | neq.astype(jnp.int32)
    out_ref[...] = xb


def _compute_sel(x):
    return pl.pallas_call(
        _sel_body,
        grid=(T // _TC_BLK,),
        in_specs=[
            pl.BlockSpec((_TC_BLK, B, F), lambda i: (i, 0, 0)),
            pl.BlockSpec((1, B, F), lambda i: (0, 0, 0)),
        ],
        out_specs=[
            pl.BlockSpec((B, F), lambda i: (0, 0)),
            pl.BlockSpec((_TC_BLK, B, F), lambda i: (i, 0, 0)),
        ],
        out_shape=[
            jax.ShapeDtypeStruct((B, F), jnp.int32),
            jax.ShapeDtypeStruct((T, B, F), jnp.float32),
        ],
        compiler_params=pltpu.CompilerParams(
            dimension_semantics=("arbitrary",),
        ),
    )(x, x)


def _sc_body(x_hbm, sel_hbm, out_hbm, selv, idxg, inb0, inb1, outb0, outb1,
             sem_in0, sem_in1, sem_out0, sem_out1):
    wid = lax.axis_index("s") * NC + lax.axis_index("c")
    t0 = wid * T_PER_W

    pltpu.sync_copy(sel_hbm, selv)

    def count_all(p, acc):
        s16 = selv[p // (F // L), pl.ds((p % (F // L)) * L, L)]
        return acc + jnp.sum(s16)

    total_sel = lax.fori_loop(0, CHUNKS, count_all, jnp.int32(0))


    @pl.when(total_sel != B * F)
    def _general():
        zeros16 = jnp.zeros((L,), jnp.float32)

        def zrow(k, _):
            inb0[B, pl.ds(k * L, L)] = zeros16
            inb1[B, pl.ds(k * L, L)] = zeros16
            return 0

        lax.fori_loop(0, F // L, zrow, 0)

        iota16 = lax.iota(jnp.int32, L)
        ones16 = jnp.ones((L,), jnp.int32)

        def per_batch(b, _):
            base = b * F

            def count_step(k, acc):
                s16 = selv[b, pl.ds(k * L, L)]
                return acc + jnp.sum(s16)

            n_sel = lax.fori_loop(0, F // L, count_step, jnp.int32(0))

            def scatter_step(k, carry):
                csel, cuns = carry
                s16 = selv[b, pl.ds(k * L, L)]
                is_sel = s16 > 0
                c = plsc.cumsum(s16)
                u = plsc.cumsum(ones16 - s16)
                pos = jnp.where(is_sel, csel + c - 1, cuns + u - 1)
                val = jnp.where(is_sel, base + k * L + iota16,
                                jnp.full((L,), ZERO_SLOT, jnp.int32))
                plsc.store_scatter(idxg, [base + pos], val)
                nsel16 = jnp.sum(s16)
                return csel + nsel16, cuns + (L - nsel16)

            lax.fori_loop(0, F // L, scatter_step, (jnp.int32(0), n_sel))
            return 0

        lax.fori_loop(0, B, per_batch, 0)

        def in_copy(t, buf, sem):
            return pltpu.make_async_copy(x_hbm.at[t], buf.at[pl.ds(0, B)],
                                         sem)

        def out_copy(t, buf, sem):
            return pltpu.make_async_copy(buf, out_hbm.at[t], sem)

        def permute(src, dst):
            @plsc.parallel_loop(0, CHUNKS, 1, unroll=8)
            def _chunk(p):
                g = idxg[pl.ds(p * L, L)]
                ir = lax.shift_right_logical(g, 9)
                ic = lax.bitwise_and(g, F - 1)
                v = plsc.load_gather(src, [ir, ic])
                dst[p // (F // L), pl.ds((p % (F // L)) * L, L)] = v

        in_copy(t0, inb0, sem_in0).start()
        in_copy(t0 + 1, inb1, sem_in1).start()

        def step(g, _):
            ta = t0 + 2 * g
            in_copy(ta, inb0, sem_in0).wait()

            @pl.when(g > 0)
            def _wait_out0():
                out_copy(ta - 2, outb0, sem_out0).wait()

            permute(inb0, outb0)

            @pl.when(g < T_PER_W // 2 - 1)
            def _next_in0():
                in_copy(ta + 2, inb0, sem_in0).start()

            out_copy(ta, outb0, sem_out0).start()

            in_copy(ta + 1, inb1, sem_in1).wait()

            @pl.when(g > 0)
            def _wait_out1():
                out_copy(ta - 1, outb1, sem_out1).wait()

            permute(inb1, outb1)

            @pl.when(g < T_PER_W // 2 - 1)
            def _next_in1():
                in_copy(ta + 3, inb1, sem_in1).start()

            out_copy(ta + 1, outb1, sem_out1).start()
            return 0

        lax.fori_loop(0, T_PER_W // 2, step, 0)

        out_copy(t0 + T_PER_W - 2, outb0, sem_out0).wait()
        out_copy(t0 + T_PER_W - 1, outb1, sem_out1).wait()


_sc_finish = functools.partial(
    pl.kernel,
    out_type=(),
    mesh=plsc.VectorSubcoreMesh(
        core_axis_name="c", subcore_axis_name="s", num_cores=NC,
        num_subcores=NS),
    compiler_params=pltpu.CompilerParams(needs_layout_passes=False),
    scratch_types=[
        pltpu.VMEM((B, F), jnp.int32),
        pltpu.VMEM((ROW,), jnp.int32),
        pltpu.VMEM((B + 1, F), jnp.float32),
        pltpu.VMEM((B + 1, F), jnp.float32),
        pltpu.VMEM((B, F), jnp.float32),
        pltpu.VMEM((B, F), jnp.float32),
        pltpu.SemaphoreType.DMA,
        pltpu.SemaphoreType.DMA,
        pltpu.SemaphoreType.DMA,
        pltpu.SemaphoreType.DMA,
    ],
)(_sc_body)


@jax.jit
def kernel(x):
    sel, out0 = _compute_sel(x)
    out_ref = jax.new_ref(out0)
    _sc_finish(x, sel, out_ref)
    return jax.freeze(out_ref)


# speedup vs baseline: 1.0483x; 1.0483x over previous
import functools

import jax
import jax.numpy as jnp
from jax import lax
from jax.experimental import pallas as pl
from jax.experimental.pallas import tpu as pltpu
from jax.experimental.pallas import tpu_sc as plsc

NC = 2
NS = 16
NW = NC * NS
L = 16

T, B, F = 4096, 16, 512
ROW = B * F
ZERO_SLOT = ROW
T_PER_W = T // NW
CHUNKS = ROW // L


_TC_BLK = 256


def _sel_body(x_ref, x0_ref, sel_ref, flag_ref, out_ref):
    @pl.when(pl.program_id(0) == 0)
    def _init():
        sel_ref[...] = jnp.zeros((B, F), jnp.int32)

    xb = x_ref[...]
    neq = (xb != x0_ref[...]).any(axis=0)
    sel_ref[...] = sel_ref[...] | neq.astype(jnp.int32)
    out_ref[...] = xb

    @pl.when(pl.program_id(0) == T // _TC_BLK - 1)
    def _flag():
        flag_ref[...] = jnp.full((8, 128), jnp.sum(sel_ref[...]), jnp.int32)


def _compute_sel(x):
    return pl.pallas_call(
        _sel_body,
        grid=(T // _TC_BLK,),
        in_specs=[
            pl.BlockSpec((_TC_BLK, B, F), lambda i: (i, 0, 0)),
            pl.BlockSpec((1, B, F), lambda i: (0, 0, 0)),
        ],
        out_specs=[
            pl.BlockSpec((B, F), lambda i: (0, 0)),
            pl.BlockSpec((8, 128), lambda i: (0, 0)),
            pl.BlockSpec((_TC_BLK, B, F), lambda i: (i, 0, 0)),
        ],
        out_shape=[
            jax.ShapeDtypeStruct((B, F), jnp.int32),
            jax.ShapeDtypeStruct((8, 128), jnp.int32),
            jax.ShapeDtypeStruct((T, B, F), jnp.float32),
        ],
        compiler_params=pltpu.CompilerParams(
            dimension_semantics=("arbitrary",),
        ),
    )(x, x)


def _sc_body(x_hbm, sel_hbm, flag_hbm, out_hbm, selv, flagv, idxg, inb0,
             inb1, outb0, outb1, sem_in0, sem_in1, sem_out0, sem_out1):
    wid = lax.axis_index("s") * NC + lax.axis_index("c")
    t0 = wid * T_PER_W

    pltpu.sync_copy(flag_hbm, flagv)
    total_sel = jnp.max(flagv[0, pl.ds(0, L)])


    @pl.when(total_sel != B * F)
    def _general():
        pltpu.sync_copy(sel_hbm, selv)
        zeros16 = jnp.zeros((L,), jnp.float32)

        def zrow(k, _):
            inb0[B, pl.ds(k * L, L)] = zeros16
            inb1[B, pl.ds(k * L, L)] = zeros16
            return 0

        lax.fori_loop(0, F // L, zrow, 0)

        iota16 = lax.iota(jnp.int32, L)
        ones16 = jnp.ones((L,), jnp.int32)

        def per_batch(b, _):
            base = b * F

            def count_step(k, acc):
                s16 = selv[b, pl.ds(k * L, L)]
                return acc + jnp.sum(s16)

            n_sel = lax.fori_loop(0, F // L, count_step, jnp.int32(0))

            def scatter_step(k, carry):
                csel, cuns = carry
                s16 = selv[b, pl.ds(k * L, L)]
                is_sel = s16 > 0
                c = plsc.cumsum(s16)
                u = plsc.cumsum(ones16 - s16)
                pos = jnp.where(is_sel, csel + c - 1, cuns + u - 1)
                val = jnp.where(is_sel, base + k * L + iota16,
                                jnp.full((L,), ZERO_SLOT, jnp.int32))
                plsc.store_scatter(idxg, [base + pos], val)
                nsel16 = jnp.sum(s16)
                return csel + nsel16, cuns + (L - nsel16)

            lax.fori_loop(0, F // L, scatter_step, (jnp.int32(0), n_sel))
            return 0

        lax.fori_loop(0, B, per_batch, 0)

        def in_copy(t, buf, sem):
            return pltpu.make_async_copy(x_hbm.at[t], buf.at[pl.ds(0, B)],
                                         sem)

        def out_copy(t, buf, sem):
            return pltpu.make_async_copy(buf, out_hbm.at[t], sem)

        def permute(src, dst):
            @plsc.parallel_loop(0, CHUNKS, 1, unroll=8)
            def _chunk(p):
                g = idxg[pl.ds(p * L, L)]
                ir = lax.shift_right_logical(g, 9)
                ic = lax.bitwise_and(g, F - 1)
                v = plsc.load_gather(src, [ir, ic])
                dst[p // (F // L), pl.ds((p % (F // L)) * L, L)] = v

        in_copy(t0, inb0, sem_in0).start()
        in_copy(t0 + 1, inb1, sem_in1).start()

        def step(g, _):
            ta = t0 + 2 * g
            in_copy(ta, inb0, sem_in0).wait()

            @pl.when(g > 0)
            def _wait_out0():
                out_copy(ta - 2, outb0, sem_out0).wait()

            permute(inb0, outb0)

            @pl.when(g < T_PER_W // 2 - 1)
            def _next_in0():
                in_copy(ta + 2, inb0, sem_in0).start()

            out_copy(ta, outb0, sem_out0).start()

            in_copy(ta + 1, inb1, sem_in1).wait()

            @pl.when(g > 0)
            def _wait_out1():
                out_copy(ta - 1, outb1, sem_out1).wait()

            permute(inb1, outb1)

            @pl.when(g < T_PER_W // 2 - 1)
            def _next_in1():
                in_copy(ta + 3, inb1, sem_in1).start()

            out_copy(ta + 1, outb1, sem_out1).start()
            return 0

        lax.fori_loop(0, T_PER_W // 2, step, 0)

        out_copy(t0 + T_PER_W - 2, outb0, sem_out0).wait()
        out_copy(t0 + T_PER_W - 1, outb1, sem_out1).wait()


_sc_finish = functools.partial(
    pl.kernel,
    out_type=(),
    mesh=plsc.VectorSubcoreMesh(
        core_axis_name="c", subcore_axis_name="s", num_cores=NC,
        num_subcores=NS),
    compiler_params=pltpu.CompilerParams(needs_layout_passes=False),
    scratch_types=[
        pltpu.VMEM((B, F), jnp.int32),
        pltpu.VMEM((8, 128), jnp.int32),
        pltpu.VMEM((ROW,), jnp.int32),
        pltpu.VMEM((B + 1, F), jnp.float32),
        pltpu.VMEM((B + 1, F), jnp.float32),
        pltpu.VMEM((B, F), jnp.float32),
        pltpu.VMEM((B, F), jnp.float32),
        pltpu.SemaphoreType.DMA,
        pltpu.SemaphoreType.DMA,
        pltpu.SemaphoreType.DMA,
        pltpu.SemaphoreType.DMA,
    ],
)(_sc_body)


@jax.jit
def kernel(x):
    sel, flag, out0 = _compute_sel(x)
    out_ref = jax.new_ref(out0)
    _sc_finish(x, sel, flag, out_ref)
    return jax.freeze(out_ref)
